# baseline (speedup 1.0000x reference)
import functools

import jax
import jax.numpy as jnp
from jax import lax
from jax.experimental import pallas as pl
from jax.experimental.pallas import tpu as pltpu
from jax.experimental.pallas import tpu_sc as plsc

SPLIT = 8
_NW = 32
_RB = 16
_W = 512
_BLK = _RB * _W


def _sc_body(nblk, pred_hbm, target_hbm, maskf_hbm, out_hbm,
             mask_v0, mask_v1, p_v0, p_v1, t_v0, t_v1, part_v, sem0, sem1):
    c = lax.axis_index("c")
    s = lax.axis_index("s")
    w = s * 2 + c
    bufs = ((mask_v0, p_v0, t_v0, sem0), (mask_v1, p_v1, t_v1, sem1))

    def fire(k, buf):
        mask_v, p_v, t_v, sem = bufs[buf]
        blk = w * nblk + k
        b = blk // 32
        h = (blk % 32) * _RB
        hs = []
        hs.append(pltpu.async_copy(
            maskf_hbm.at[pl.ds(b * _W + h, _RB), :], mask_v, sem))
        for ch in range(3):
            prow = (b * 3 + ch) * _W + h
            hs.append(pltpu.async_copy(
                pred_hbm.at[pl.ds(prow, _RB), :],
                p_v.at[pl.ds(ch * _RB, _RB), :], sem))
            hs.append(pltpu.async_copy(
                target_hbm.at[pl.ds(prow, _RB), :],
                t_v.at[pl.ds(ch * _RB, _RB), :], sem))
        return hs

    def block_accum(buf, acc):
        mask_v, p_v, t_v, _ = bufs[buf]

        def body(g, carry):
            a_s, a_k = carry
            r = g // (_W // 16)
            o = (g % (_W // 16)) * 16
            m = 1.0 - mask_v[r, pl.ds(o, 16)]
            d0 = t_v[r, pl.ds(o, 16)] - p_v[r, pl.ds(o, 16)]
            d1 = (t_v[_RB + r, pl.ds(o, 16)] - p_v[_RB + r, pl.ds(o, 16)])
            d2 = (t_v[2 * _RB + r, pl.ds(o, 16)] - p_v[2 * _RB + r, pl.ds(o, 16)])
            a_s = a_s + (d0 * d0 + d1 * d1 + d2 * d2) * m
            a_k = a_k + m
            return (a_s, a_k)
        return lax.fori_loop(0, _BLK // 16, body, acc)

    acc = (jnp.zeros((16,), jnp.float32), jnp.zeros((16,), jnp.float32))
    pending = fire(0, 0)
    for k in range(nblk):
        nxt = fire(k + 1, (k + 1) % 2) if k + 1 < nblk else []
        for h in pending:
            h.wait()
        acc = block_accum(k % 2, acc)
        pending = nxt

    part_v[pl.ds(0, 16)] = acc[0]
    part_v[pl.ds(16, 16)] = acc[1]
    pltpu.sync_copy(part_v, out_hbm.at[pl.ds(w * 32, 32)])


def _tc_body(pred_ref, target_ref, mask_ref, out_ref, acc_ref):
    i = pl.program_id(0)

    @pl.when(i == 0)
    def _init():
        acc_ref[0] = 0.0
        acc_ref[1] = 0.0

    kf = 1.0 - mask_ref[0, 0].astype(jnp.float32)
    d = target_ref[0] - pred_ref[0]
    acc_ref[0] += jnp.sum(d * d * kf[None, :, :])
    acc_ref[1] += jnp.sum(kf) * 3.0

    @pl.when(i == pl.num_programs(0) - 1)
    def _fin():
        out_ref[0] = acc_ref[0]
        out_ref[1] = acc_ref[1]


def _combine_body(sc_ref, tc_ref, out_ref):
    s = jnp.sum(sc_ref[:, 0, :]) + tc_ref[0]
    cnt = 3.0 * jnp.sum(sc_ref[:, 1, :]) + tc_ref[1]
    out_ref[0] = s / cnt


def kernel(pred, target, sky_mask):
    B, C, H, W = pred.shape
    bs = B - SPLIT
    nblk = bs * (H // _RB) // _NW

    maskf = sky_mask[SPLIT:].astype(jnp.float32).reshape(-1, W)
    sc_parts = pl.kernel(
        functools.partial(_sc_body, nblk),
        out_type=jax.ShapeDtypeStruct((_NW * 32,), jnp.float32),
        mesh=plsc.VectorSubcoreMesh(core_axis_name="c", subcore_axis_name="s"),
        compiler_params=pltpu.CompilerParams(use_tc_tiling_on_sc=True),
        scratch_types=[
            pltpu.VMEM((_RB, _W), jnp.float32),
            pltpu.VMEM((_RB, _W), jnp.float32),
            pltpu.VMEM((3 * _RB, _W), jnp.float32),
            pltpu.VMEM((3 * _RB, _W), jnp.float32),
            pltpu.VMEM((3 * _RB, _W), jnp.float32),
            pltpu.VMEM((3 * _RB, _W), jnp.float32),
            pltpu.VMEM((32,), jnp.float32),
            pltpu.SemaphoreType.DMA,
            pltpu.SemaphoreType.DMA,
        ],
    )(pred[SPLIT:].reshape(-1, W), target[SPLIT:].reshape(-1, W), maskf)
    sc_parts = sc_parts.reshape(_NW, 2, 16)

    if SPLIT > 0:
        tc_parts = pl.pallas_call(
            _tc_body,
            grid=(SPLIT,),
            in_specs=[
                pl.BlockSpec((1, C, H, W), lambda i: (i, 0, 0, 0)),
                pl.BlockSpec((1, C, H, W), lambda i: (i, 0, 0, 0)),
                pl.BlockSpec((1, 1, H, W), lambda i: (i, 0, 0, 0)),
            ],
            out_specs=pl.BlockSpec(memory_space=pltpu.SMEM),
            out_shape=jax.ShapeDtypeStruct((2,), jnp.float32),
            scratch_shapes=[pltpu.SMEM((2,), jnp.float32)],
        )(pred[:SPLIT], target[:SPLIT], sky_mask[:SPLIT])
    else:
        tc_parts = jnp.zeros((2,), jnp.float32)

    out = pl.pallas_call(
        _combine_body,
        in_specs=[
            pl.BlockSpec((_NW, 2, 16), lambda: (0, 0, 0)),
            pl.BlockSpec(memory_space=pltpu.SMEM),
        ],
        out_specs=pl.BlockSpec(memory_space=pltpu.SMEM),
        out_shape=jax.ShapeDtypeStruct((1,), jnp.float32),
    )(sc_parts, tc_parts)
    return out[0]


# speedup vs baseline: 1.9138x; 1.9138x over previous
import functools

import jax
import jax.numpy as jnp
from jax import lax
from jax.experimental import pallas as pl
from jax.experimental.pallas import tpu as pltpu
from jax.experimental.pallas import tpu_sc as plsc

SPLIT = 8
_NW = 32
_RB = 16
_W = 512
_BLK = _RB * _W


def _sc_body(nblk, pred_hbm, target_hbm, maskf_hbm, out_hbm,
             mask_v0, mask_v1, p_v0, p_v1, t_v0, t_v1, part_v, sem0, sem1):
    c = lax.axis_index("c")
    s = lax.axis_index("s")
    w = s * 2 + c
    bufs = ((mask_v0, p_v0, t_v0, sem0), (mask_v1, p_v1, t_v1, sem1))

    def fire(k, buf):
        mask_v, p_v, t_v, sem = bufs[buf]
        blk = w * nblk + k
        b = blk // 32
        h = (blk % 32) * _RB
        hs = []
        hs.append(pltpu.async_copy(
            maskf_hbm.at[pl.ds(b * _W + h, _RB), :], mask_v, sem))
        for ch in range(3):
            prow = ((SPLIT + b) * 3 + ch) * _W + h
            hs.append(pltpu.async_copy(
                pred_hbm.at[pl.ds(prow, _RB), :],
                p_v.at[pl.ds(ch * _RB, _RB), :], sem))
            hs.append(pltpu.async_copy(
                target_hbm.at[pl.ds(prow, _RB), :],
                t_v.at[pl.ds(ch * _RB, _RB), :], sem))
        return hs

    def block_accum(buf, acc):
        mask_v, p_v, t_v, _ = bufs[buf]

        def body(g, carry):
            a_s, a_k = carry
            r = g // (_W // 16)
            o = (g % (_W // 16)) * 16
            m = 1.0 - mask_v[r, pl.ds(o, 16)]
            d0 = t_v[r, pl.ds(o, 16)] - p_v[r, pl.ds(o, 16)]
            d1 = (t_v[_RB + r, pl.ds(o, 16)] - p_v[_RB + r, pl.ds(o, 16)])
            d2 = (t_v[2 * _RB + r, pl.ds(o, 16)] - p_v[2 * _RB + r, pl.ds(o, 16)])
            a_s = a_s + (d0 * d0 + d1 * d1 + d2 * d2) * m
            a_k = a_k + m
            return (a_s, a_k)
        return lax.fori_loop(0, _BLK // 16, body, acc)

    acc = (jnp.zeros((16,), jnp.float32), jnp.zeros((16,), jnp.float32))
    pending = fire(0, 0)
    for k in range(nblk):
        nxt = fire(k + 1, (k + 1) % 2) if k + 1 < nblk else []
        for h in pending:
            h.wait()
        acc = block_accum(k % 2, acc)
        pending = nxt

    part_v[pl.ds(0, 16)] = acc[0]
    part_v[pl.ds(16, 16)] = acc[1]
    pltpu.sync_copy(part_v, out_hbm.at[pl.ds(w * 32, 32)])


def _tc_body(pred_ref, target_ref, mask_ref, out_ref, acc_ref):
    i = pl.program_id(0)

    @pl.when(i == 0)
    def _init():
        acc_ref[0] = 0.0
        acc_ref[1] = 0.0

    kf = 1.0 - mask_ref[0, 0].astype(jnp.float32)
    d = target_ref[0] - pred_ref[0]
    acc_ref[0] += jnp.sum(d * d * kf[None, :, :])
    acc_ref[1] += jnp.sum(kf) * 3.0

    @pl.when(i == pl.num_programs(0) - 1)
    def _fin():
        out_ref[0] = acc_ref[0]
        out_ref[1] = acc_ref[1]


def _combine_body(sc_ref, tc_ref, out_ref):
    s = jnp.sum(sc_ref[:, 0, :]) + tc_ref[0]
    cnt = 3.0 * jnp.sum(sc_ref[:, 1, :]) + tc_ref[1]
    out_ref[0] = s / cnt


def kernel(pred, target, sky_mask):
    B, C, H, W = pred.shape
    bs = B - SPLIT
    nblk = bs * (H // _RB) // _NW

    maskf = sky_mask[SPLIT:].astype(jnp.float32).reshape(-1, W)
    sc_parts = pl.kernel(
        functools.partial(_sc_body, nblk),
        out_type=jax.ShapeDtypeStruct((_NW * 32,), jnp.float32),
        mesh=plsc.VectorSubcoreMesh(core_axis_name="c", subcore_axis_name="s"),
        compiler_params=pltpu.CompilerParams(use_tc_tiling_on_sc=True),
        scratch_types=[
            pltpu.VMEM((_RB, _W), jnp.float32),
            pltpu.VMEM((_RB, _W), jnp.float32),
            pltpu.VMEM((3 * _RB, _W), jnp.float32),
            pltpu.VMEM((3 * _RB, _W), jnp.float32),
            pltpu.VMEM((3 * _RB, _W), jnp.float32),
            pltpu.VMEM((3 * _RB, _W), jnp.float32),
            pltpu.VMEM((32,), jnp.float32),
            pltpu.SemaphoreType.DMA,
            pltpu.SemaphoreType.DMA,
        ],
    )(pred.reshape(-1, W), target.reshape(-1, W), maskf)
    sc_parts = sc_parts.reshape(_NW, 2, 16)

    if SPLIT > 0:
        tc_parts = pl.pallas_call(
            _tc_body,
            grid=(SPLIT,),
            in_specs=[
                pl.BlockSpec((1, C, H, W), lambda i: (i, 0, 0, 0)),
                pl.BlockSpec((1, C, H, W), lambda i: (i, 0, 0, 0)),
                pl.BlockSpec((1, 1, H, W), lambda i: (i, 0, 0, 0)),
            ],
            out_specs=pl.BlockSpec(memory_space=pltpu.SMEM),
            out_shape=jax.ShapeDtypeStruct((2,), jnp.float32),
            scratch_shapes=[pltpu.SMEM((2,), jnp.float32)],
        )(pred, target, sky_mask)
    else:
        tc_parts = jnp.zeros((2,), jnp.float32)

    out = pl.pallas_call(
        _combine_body,
        in_specs=[
            pl.BlockSpec((_NW, 2, 16), lambda: (0, 0, 0)),
            pl.BlockSpec(memory_space=pltpu.SMEM),
        ],
        out_specs=pl.BlockSpec(memory_space=pltpu.SMEM),
        out_shape=jax.ShapeDtypeStruct((1,), jnp.float32),
    )(sc_parts, tc_parts)
    return out[0]


# speedup vs baseline: 2.0181x; 1.0545x over previous
import functools

import jax
import jax.numpy as jnp
from jax import lax
from jax.experimental import pallas as pl
from jax.experimental.pallas import tpu as pltpu
from jax.experimental.pallas import tpu_sc as plsc

SPLIT = 8
_NW = 32
_RB = 16
_W = 512
_BLK = _RB * _W


def _sc_body(nblk, pred_hbm, target_hbm, maskf_hbm, out_hbm,
             mask_v0, mask_v1, p_v0, p_v1, t_v0, t_v1, part_v, sem0, sem1):
    c = lax.axis_index("c")
    s = lax.axis_index("s")
    w = s * 2 + c
    bufs = ((mask_v0, p_v0, t_v0, sem0), (mask_v1, p_v1, t_v1, sem1))

    def fire(k, buf):
        mask_v, p_v, t_v, sem = bufs[buf]
        blk = w * nblk + k
        b = blk // 32
        h = (blk % 32) * _RB
        hs = []
        hs.append(pltpu.async_copy(
            maskf_hbm.at[pl.ds(b * _W + h, _RB), :], mask_v, sem))
        for ch in range(3):
            prow = ((SPLIT + b) * 3 + ch) * _W + h
            hs.append(pltpu.async_copy(
                pred_hbm.at[pl.ds(prow, _RB), :],
                p_v.at[pl.ds(ch * _RB, _RB), :], sem))
            hs.append(pltpu.async_copy(
                target_hbm.at[pl.ds(prow, _RB), :],
                t_v.at[pl.ds(ch * _RB, _RB), :], sem))
        return hs

    def block_accum(buf, acc):
        mask_v, p_v, t_v, _ = bufs[buf]

        def body(g, carry):
            a_s, a_k = carry
            r = g // (_W // 16)
            o = (g % (_W // 16)) * 16
            m = 1.0 - mask_v[r, pl.ds(o, 16)]
            d0 = t_v[r, pl.ds(o, 16)] - p_v[r, pl.ds(o, 16)]
            d1 = (t_v[_RB + r, pl.ds(o, 16)] - p_v[_RB + r, pl.ds(o, 16)])
            d2 = (t_v[2 * _RB + r, pl.ds(o, 16)] - p_v[2 * _RB + r, pl.ds(o, 16)])
            a_s = a_s + (d0 * d0 + d1 * d1 + d2 * d2) * m
            a_k = a_k + m
            return (a_s, a_k)
        return lax.fori_loop(0, _BLK // 16, body, acc)

    acc = (jnp.zeros((16,), jnp.float32), jnp.zeros((16,), jnp.float32))
    pending = fire(0, 0)
    for k in range(nblk):
        nxt = fire(k + 1, (k + 1) % 2) if k + 1 < nblk else []
        for h in pending:
            h.wait()
        acc = block_accum(k % 2, acc)
        pending = nxt

    part_v[pl.ds(0, 16)] = acc[0]
    part_v[pl.ds(16, 16)] = acc[1]
    pltpu.sync_copy(part_v, out_hbm.at[pl.ds(w * 32, 32)])


def _tc_body(pred_ref, target_ref, mask_ref, out_ref, acc_ref):
    i = pl.program_id(0)

    @pl.when(i == 0)
    def _init():
        acc_ref[0] = 0.0
        acc_ref[1] = 0.0

    kf = 1.0 - mask_ref[0, 0].astype(jnp.float32)
    d = target_ref[0] - pred_ref[0]
    acc_ref[0] += jnp.sum(d * d * kf[None, :, :])
    acc_ref[1] += jnp.sum(kf) * 3.0

    @pl.when(i == pl.num_programs(0) - 1)
    def _fin():
        out_ref[0] = acc_ref[0]
        out_ref[1] = acc_ref[1]


def _combine_body(sc_ref, tc_ref, out_ref):
    s = jnp.sum(sc_ref[:, 0, :]) + tc_ref[0]
    cnt = 3.0 * jnp.sum(sc_ref[:, 1, :]) + tc_ref[1]
    out_ref[0] = s / cnt


def kernel(pred, target, sky_mask):
    B, C, H, W = pred.shape
    bs = B - SPLIT
    nblk = bs * (H // _RB) // _NW

    maskf = sky_mask[SPLIT:].astype(jnp.float32).reshape(-1, W)
    sc_parts = pl.kernel(
        functools.partial(_sc_body, nblk),
        out_type=jax.ShapeDtypeStruct((_NW * 32,), jnp.float32),
        mesh=plsc.VectorSubcoreMesh(core_axis_name="c", subcore_axis_name="s"),
        compiler_params=pltpu.CompilerParams(use_tc_tiling_on_sc=True),
        scratch_types=[
            pltpu.VMEM((_RB, _W), jnp.float32),
            pltpu.VMEM((_RB, _W), jnp.float32),
            pltpu.VMEM((3 * _RB, _W), jnp.float32),
            pltpu.VMEM((3 * _RB, _W), jnp.float32),
            pltpu.VMEM((3 * _RB, _W), jnp.float32),
            pltpu.VMEM((3 * _RB, _W), jnp.float32),
            pltpu.VMEM((32,), jnp.float32),
            pltpu.SemaphoreType.DMA,
            pltpu.SemaphoreType.DMA,
        ],
    )(pred.reshape(-1, W), target.reshape(-1, W), maskf)
    sc_parts = sc_parts.reshape(_NW, 2, 16)

    if SPLIT > 0:
        tc_parts = pl.pallas_call(
            _tc_body,
            grid=(SPLIT,),
            in_specs=[
                pl.BlockSpec((1, C, H, W), lambda i: (i, 0, 0, 0)),
                pl.BlockSpec((1, C, H, W), lambda i: (i, 0, 0, 0)),
                pl.BlockSpec((1, 1, H, W), lambda i: (i, 0, 0, 0)),
            ],
            out_specs=pl.BlockSpec(memory_space=pltpu.SMEM),
            out_shape=jax.ShapeDtypeStruct((2,), jnp.float32),
            scratch_shapes=[pltpu.SMEM((2,), jnp.float32)],
        )(pred, target, sky_mask.view(jnp.uint8))
    else:
        tc_parts = jnp.zeros((2,), jnp.float32)

    out = pl.pallas_call(
        _combine_body,
        in_specs=[
            pl.BlockSpec((_NW, 2, 16), lambda: (0, 0, 0)),
            pl.BlockSpec(memory_space=pltpu.SMEM),
        ],
        out_specs=pl.BlockSpec(memory_space=pltpu.SMEM),
        out_shape=jax.ShapeDtypeStruct((1,), jnp.float32),
    )(sc_parts, tc_parts)
    return out[0]


# speedup vs baseline: 2.0412x; 1.0115x over previous
import functools

import jax
import jax.numpy as jnp
from jax import lax
from jax.experimental import pallas as pl
from jax.experimental.pallas import tpu as pltpu
from jax.experimental.pallas import tpu_sc as plsc

SPLIT = 8
_NW = 32
_RB = 16
_W = 512
_BLK = _RB * _W


def _sc_body(nblk, pred_hbm, target_hbm, maskf_hbm, out_hbm,
             mask_v0, mask_v1, p_v0, p_v1, t_v0, t_v1, part_v, sem0, sem1):
    c = lax.axis_index("c")
    s = lax.axis_index("s")
    w = s * 2 + c
    bufs = ((mask_v0, p_v0, t_v0, sem0), (mask_v1, p_v1, t_v1, sem1))

    def fire(k, buf):
        mask_v, p_v, t_v, sem = bufs[buf]
        blk = w * nblk + k
        b = blk // 32
        h = (blk % 32) * _RB
        hs = []
        hs.append(pltpu.async_copy(
            maskf_hbm.at[pl.ds(b * _W + h, _RB), :], mask_v, sem))
        for ch in range(3):
            prow = ((SPLIT + b) * 3 + ch) * _W + h
            hs.append(pltpu.async_copy(
                pred_hbm.at[pl.ds(prow, _RB), :],
                p_v.at[pl.ds(ch * _RB, _RB), :], sem))
            hs.append(pltpu.async_copy(
                target_hbm.at[pl.ds(prow, _RB), :],
                t_v.at[pl.ds(ch * _RB, _RB), :], sem))
        return hs

    def block_accum(buf, acc):
        mask_v, p_v, t_v, _ = bufs[buf]

        def body(g, carry):
            accs = list(carry[:2])
            ks = list(carry[2:])
            r = g // 4
            for j in range(8):
                o = (g % 4) * 128 + j * 16
                lane = j % 2
                m = 1.0 - mask_v[r, pl.ds(o, 16)]
                d0 = t_v[r, pl.ds(o, 16)] - p_v[r, pl.ds(o, 16)]
                d1 = (t_v[_RB + r, pl.ds(o, 16)]
                      - p_v[_RB + r, pl.ds(o, 16)])
                d2 = (t_v[2 * _RB + r, pl.ds(o, 16)]
                      - p_v[2 * _RB + r, pl.ds(o, 16)])
                accs[lane] = accs[lane] + (d0 * d0 + d1 * d1 + d2 * d2) * m
                ks[lane] = ks[lane] + m
            return (*accs, *ks)
        return lax.fori_loop(0, _RB * 4, body, acc)

    z = jnp.zeros((16,), jnp.float32)
    acc = (z, z, z, z)
    pending = fire(0, 0)
    for k in range(nblk):
        nxt = fire(k + 1, (k + 1) % 2) if k + 1 < nblk else []
        for h in pending:
            h.wait()
        acc = block_accum(k % 2, acc)
        pending = nxt

    part_v[pl.ds(0, 16)] = acc[0] + acc[1]
    part_v[pl.ds(16, 16)] = acc[2] + acc[3]
    pltpu.sync_copy(part_v.at[pl.ds(0, 16)], out_hbm.at[pl.ds(w * 16, 16)])
    pltpu.sync_copy(part_v.at[pl.ds(16, 16)],
                    out_hbm.at[pl.ds(_NW * 16 + w * 16, 16)])


def _tc_body(pred_ref, target_ref, mask_ref, out_ref, acc_ref):
    i = pl.program_id(0)

    @pl.when(i == 0)
    def _init():
        acc_ref[0] = 0.0
        acc_ref[1] = 0.0

    kf = 1.0 - mask_ref[0, 0].astype(jnp.float32)
    d = target_ref[0] - pred_ref[0]
    acc_ref[0] += jnp.sum(d * d * kf[None, :, :])
    acc_ref[1] += jnp.sum(kf) * 3.0

    @pl.when(i == pl.num_programs(0) - 1)
    def _fin():
        out_ref[0] = acc_ref[0]
        out_ref[1] = acc_ref[1]


def _combine_body(sc_ref, tc_ref, out_ref):
    s = jnp.sum(sc_ref[pl.ds(0, _NW * 16)]) + tc_ref[0]
    cnt = 3.0 * jnp.sum(sc_ref[pl.ds(_NW * 16, _NW * 16)]) + tc_ref[1]
    out_ref[0] = s / cnt


def kernel(pred, target, sky_mask):
    B, C, H, W = pred.shape
    bs = B - SPLIT
    nblk = bs * (H // _RB) // _NW

    maskf = sky_mask[SPLIT:].astype(jnp.float32).reshape(-1, W)
    sc_parts = pl.kernel(
        functools.partial(_sc_body, nblk),
        out_type=jax.ShapeDtypeStruct((_NW * 32,), jnp.float32),
        mesh=plsc.VectorSubcoreMesh(core_axis_name="c", subcore_axis_name="s"),
        compiler_params=pltpu.CompilerParams(use_tc_tiling_on_sc=True),
        scratch_types=[
            pltpu.VMEM((_RB, _W), jnp.float32),
            pltpu.VMEM((_RB, _W), jnp.float32),
            pltpu.VMEM((3 * _RB, _W), jnp.float32),
            pltpu.VMEM((3 * _RB, _W), jnp.float32),
            pltpu.VMEM((3 * _RB, _W), jnp.float32),
            pltpu.VMEM((3 * _RB, _W), jnp.float32),
            pltpu.VMEM((32,), jnp.float32),
            pltpu.SemaphoreType.DMA,
            pltpu.SemaphoreType.DMA,
        ],
    )(pred.reshape(-1, W), target.reshape(-1, W), maskf)

    if SPLIT > 0:
        tc_parts = pl.pallas_call(
            _tc_body,
            grid=(SPLIT,),
            in_specs=[
                pl.BlockSpec((1, C, H, W), lambda i: (i, 0, 0, 0)),
                pl.BlockSpec((1, C, H, W), lambda i: (i, 0, 0, 0)),
                pl.BlockSpec((1, 1, H, W), lambda i: (i, 0, 0, 0)),
            ],
            out_specs=pl.BlockSpec(memory_space=pltpu.SMEM),
            out_shape=jax.ShapeDtypeStruct((2,), jnp.float32),
            scratch_shapes=[pltpu.SMEM((2,), jnp.float32)],
        )(pred, target, sky_mask.view(jnp.uint8))
    else:
        tc_parts = jnp.zeros((2,), jnp.float32)

    out = pl.pallas_call(
        _combine_body,
        in_specs=[
            pl.BlockSpec((_NW * 32,), lambda: (0,)),
            pl.BlockSpec(memory_space=pltpu.SMEM),
        ],
        out_specs=pl.BlockSpec(memory_space=pltpu.SMEM),
        out_shape=jax.ShapeDtypeStruct((1,), jnp.float32),
    )(sc_parts, tc_parts)
    return out[0]


# speedup vs baseline: 2.2477x; 1.1012x over previous
import functools

import jax
import jax.numpy as jnp
from jax import lax
from jax.experimental import pallas as pl
from jax.experimental.pallas import tpu as pltpu
from jax.experimental.pallas import tpu_sc as plsc

SPLIT = 12
_NW = 32
_RB = 16
_W = 512
_BLK = _RB * _W


def _sc_body(nblk, pred_hbm, target_hbm, maskf_hbm, out_hbm,
             mask_v0, mask_v1, p_v0, p_v1, t_v0, t_v1, part_v, sem0, sem1):
    c = lax.axis_index("c")
    s = lax.axis_index("s")
    w = s * 2 + c
    bufs = ((mask_v0, p_v0, t_v0, sem0), (mask_v1, p_v1, t_v1, sem1))

    def fire(k, buf):
        mask_v, p_v, t_v, sem = bufs[buf]
        blk = w * nblk + k
        b = blk // 32
        h = (blk % 32) * _RB
        hs = []
        hs.append(pltpu.async_copy(
            maskf_hbm.at[pl.ds(b * _W + h, _RB), :], mask_v, sem))
        for ch in range(3):
            prow = ((SPLIT + b) * 3 + ch) * _W + h
            hs.append(pltpu.async_copy(
                pred_hbm.at[pl.ds(prow, _RB), :],
                p_v.at[pl.ds(ch * _RB, _RB), :], sem))
            hs.append(pltpu.async_copy(
                target_hbm.at[pl.ds(prow, _RB), :],
                t_v.at[pl.ds(ch * _RB, _RB), :], sem))
        return hs

    def block_accum(buf, acc):
        mask_v, p_v, t_v, _ = bufs[buf]

        def body(g, carry):
            accs = list(carry[:2])
            ks = list(carry[2:])
            r = g // 4
            for j in range(8):
                o = (g % 4) * 128 + j * 16
                lane = j % 2
                m = 1.0 - mask_v[r, pl.ds(o, 16)]
                d0 = t_v[r, pl.ds(o, 16)] - p_v[r, pl.ds(o, 16)]
                d1 = (t_v[_RB + r, pl.ds(o, 16)]
                      - p_v[_RB + r, pl.ds(o, 16)])
                d2 = (t_v[2 * _RB + r, pl.ds(o, 16)]
                      - p_v[2 * _RB + r, pl.ds(o, 16)])
                accs[lane] = accs[lane] + (d0 * d0 + d1 * d1 + d2 * d2) * m
                ks[lane] = ks[lane] + m
            return (*accs, *ks)
        return lax.fori_loop(0, _RB * 4, body, acc)

    z = jnp.zeros((16,), jnp.float32)
    acc = (z, z, z, z)
    pending = fire(0, 0)
    for k in range(nblk):
        nxt = fire(k + 1, (k + 1) % 2) if k + 1 < nblk else []
        for h in pending:
            h.wait()
        acc = block_accum(k % 2, acc)
        pending = nxt

    part_v[pl.ds(0, 16)] = acc[0] + acc[1]
    part_v[pl.ds(16, 16)] = acc[2] + acc[3]
    pltpu.sync_copy(part_v.at[pl.ds(0, 16)], out_hbm.at[pl.ds(w * 16, 16)])
    pltpu.sync_copy(part_v.at[pl.ds(16, 16)],
                    out_hbm.at[pl.ds(_NW * 16 + w * 16, 16)])


def _tc_body(pred_ref, target_ref, mask_ref, out_ref, acc_ref):
    i = pl.program_id(0)

    @pl.when(i == 0)
    def _init():
        acc_ref[0] = 0.0
        acc_ref[1] = 0.0

    kf = 1.0 - mask_ref[0, 0].astype(jnp.float32)
    d = target_ref[0] - pred_ref[0]
    acc_ref[0] += jnp.sum(d * d * kf[None, :, :])
    acc_ref[1] += jnp.sum(kf) * 3.0

    @pl.when(i == pl.num_programs(0) - 1)
    def _fin():
        out_ref[0] = acc_ref[0]
        out_ref[1] = acc_ref[1]


def _combine_body(sc_ref, tc_ref, out_ref):
    s = jnp.sum(sc_ref[pl.ds(0, _NW * 16)]) + tc_ref[0]
    cnt = 3.0 * jnp.sum(sc_ref[pl.ds(_NW * 16, _NW * 16)]) + tc_ref[1]
    out_ref[0] = s / cnt


def kernel(pred, target, sky_mask):
    B, C, H, W = pred.shape
    bs = B - SPLIT
    nblk = bs * (H // _RB) // _NW

    maskf = sky_mask[SPLIT:].astype(jnp.float32).reshape(-1, W)
    sc_parts = pl.kernel(
        functools.partial(_sc_body, nblk),
        out_type=jax.ShapeDtypeStruct((_NW * 32,), jnp.float32),
        mesh=plsc.VectorSubcoreMesh(core_axis_name="c", subcore_axis_name="s"),
        compiler_params=pltpu.CompilerParams(use_tc_tiling_on_sc=True),
        scratch_types=[
            pltpu.VMEM((_RB, _W), jnp.float32),
            pltpu.VMEM((_RB, _W), jnp.float32),
            pltpu.VMEM((3 * _RB, _W), jnp.float32),
            pltpu.VMEM((3 * _RB, _W), jnp.float32),
            pltpu.VMEM((3 * _RB, _W), jnp.float32),
            pltpu.VMEM((3 * _RB, _W), jnp.float32),
            pltpu.VMEM((32,), jnp.float32),
            pltpu.SemaphoreType.DMA,
            pltpu.SemaphoreType.DMA,
        ],
    )(pred.reshape(-1, W), target.reshape(-1, W), maskf)

    if SPLIT > 0:
        tc_parts = pl.pallas_call(
            _tc_body,
            grid=(SPLIT,),
            in_specs=[
                pl.BlockSpec((1, C, H, W), lambda i: (i, 0, 0, 0)),
                pl.BlockSpec((1, C, H, W), lambda i: (i, 0, 0, 0)),
                pl.BlockSpec((1, 1, H, W), lambda i: (i, 0, 0, 0)),
            ],
            out_specs=pl.BlockSpec(memory_space=pltpu.SMEM),
            out_shape=jax.ShapeDtypeStruct((2,), jnp.float32),
            scratch_shapes=[pltpu.SMEM((2,), jnp.float32)],
        )(pred, target, sky_mask.view(jnp.uint8))
    else:
        tc_parts = jnp.zeros((2,), jnp.float32)

    out = pl.pallas_call(
        _combine_body,
        in_specs=[
            pl.BlockSpec((_NW * 32,), lambda: (0,)),
            pl.BlockSpec(memory_space=pltpu.SMEM),
        ],
        out_specs=pl.BlockSpec(memory_space=pltpu.SMEM),
        out_shape=jax.ShapeDtypeStruct((1,), jnp.float32),
    )(sc_parts, tc_parts)
    return out[0]


# speedup vs baseline: 2.4120x; 1.0731x over previous
import functools

import jax
import jax.numpy as jnp
from jax import lax
from jax.experimental import pallas as pl
from jax.experimental.pallas import tpu as pltpu
from jax.experimental.pallas import tpu_sc as plsc

SPLIT = 12
_NW = 32
_RB = 16
_W = 512
_BLK = _RB * _W


def _sc_body(nblk, pred_hbm, target_hbm, maskf_hbm, out_hbm,
             mask_v0, mask_v1, p_v0, p_v1, t_v0, t_v1, part_v, sem0, sem1):
    c = lax.axis_index("c")
    s = lax.axis_index("s")
    w = s * 2 + c
    bufs = ((mask_v0, p_v0, t_v0, sem0), (mask_v1, p_v1, t_v1, sem1))

    def fire(k, buf):
        mask_v, p_v, t_v, sem = bufs[buf]
        blk = w * nblk + k
        b = blk // 32
        h = (blk % 32) * _RB
        hs = []
        hs.append(pltpu.async_copy(
            maskf_hbm.at[pl.ds(b * _W + h, _RB), :], mask_v, sem))
        for ch in range(3):
            prow = ((SPLIT + b) * 3 + ch) * _W + h
            hs.append(pltpu.async_copy(
                pred_hbm.at[pl.ds(prow, _RB), :],
                p_v.at[pl.ds(ch * _RB, _RB), :], sem))
            hs.append(pltpu.async_copy(
                target_hbm.at[pl.ds(prow, _RB), :],
                t_v.at[pl.ds(ch * _RB, _RB), :], sem))
        return hs

    def block_accum(buf, acc):
        mask_v, p_v, t_v, _ = bufs[buf]

        def body(g, carry):
            accs = list(carry[:2])
            ks = list(carry[2:])
            r = g // 4
            for j in range(8):
                o = (g % 4) * 128 + j * 16
                lane = j % 2
                m = 1.0 - mask_v[r, pl.ds(o, 16)]
                d0 = t_v[r, pl.ds(o, 16)] - p_v[r, pl.ds(o, 16)]
                d1 = (t_v[_RB + r, pl.ds(o, 16)]
                      - p_v[_RB + r, pl.ds(o, 16)])
                d2 = (t_v[2 * _RB + r, pl.ds(o, 16)]
                      - p_v[2 * _RB + r, pl.ds(o, 16)])
                accs[lane] = accs[lane] + (d0 * d0 + d1 * d1 + d2 * d2) * m
                ks[lane] = ks[lane] + m
            return (*accs, *ks)
        return lax.fori_loop(0, _RB * 4, body, acc)

    z = jnp.zeros((16,), jnp.float32)
    acc = (z, z, z, z)
    pending = fire(0, 0)
    for k in range(nblk):
        nxt = fire(k + 1, (k + 1) % 2) if k + 1 < nblk else []
        for h in pending:
            h.wait()
        acc = block_accum(k % 2, acc)
        pending = nxt

    part_v[pl.ds(0, 16)] = acc[0] + acc[1]
    part_v[pl.ds(16, 16)] = acc[2] + acc[3]
    pltpu.sync_copy(part_v.at[pl.ds(0, 16)], out_hbm.at[pl.ds(w * 16, 16)])
    pltpu.sync_copy(part_v.at[pl.ds(16, 16)],
                    out_hbm.at[pl.ds(_NW * 16 + w * 16, 16)])


def _tc_body(pred_ref, target_ref, mask_ref, out_ref, acc_ref):
    i = pl.program_id(0)

    @pl.when(i == 0)
    def _init():
        acc_ref[0] = 0.0
        acc_ref[1] = 0.0

    kf = 1.0 - mask_ref[0, 0].astype(jnp.float32)
    d = target_ref[0] - pred_ref[0]
    acc_ref[0] += jnp.sum(d * d * kf[None, :, :])
    acc_ref[1] += jnp.sum(kf) * 3.0

    @pl.when(i == pl.num_programs(0) - 1)
    def _fin():
        out_ref[0] = acc_ref[0]
        out_ref[1] = acc_ref[1]


def _combine_body(sc_ref, tc_ref, out_ref):
    s = jnp.sum(sc_ref[pl.ds(0, _NW * 16)]) + tc_ref[0]
    cnt = 3.0 * jnp.sum(sc_ref[pl.ds(_NW * 16, _NW * 16)]) + tc_ref[1]
    out_ref[0] = s / cnt


def kernel(pred, target, sky_mask):
    B, C, H, W = pred.shape
    bs = B - SPLIT
    nblk = bs * (H // _RB) // _NW

    maskf = sky_mask[SPLIT:].astype(jnp.float32).reshape(-1, W)
    sc_parts = pl.kernel(
        functools.partial(_sc_body, nblk),
        out_type=jax.ShapeDtypeStruct((_NW * 32,), jnp.float32),
        mesh=plsc.VectorSubcoreMesh(core_axis_name="c", subcore_axis_name="s"),
        compiler_params=pltpu.CompilerParams(use_tc_tiling_on_sc=True),
        scratch_types=[
            pltpu.VMEM((_RB, _W), jnp.float32),
            pltpu.VMEM((_RB, _W), jnp.float32),
            pltpu.VMEM((3 * _RB, _W), jnp.float32),
            pltpu.VMEM((3 * _RB, _W), jnp.float32),
            pltpu.VMEM((3 * _RB, _W), jnp.float32),
            pltpu.VMEM((3 * _RB, _W), jnp.float32),
            pltpu.VMEM((32,), jnp.float32),
            pltpu.SemaphoreType.DMA,
            pltpu.SemaphoreType.DMA,
        ],
    )(pred.reshape(-1, W), target.reshape(-1, W), maskf)

    if SPLIT > 0:
        tc_parts = pl.pallas_call(
            _tc_body,
            grid=(SPLIT,),
            compiler_params=pltpu.CompilerParams(
                allow_input_fusion=[False, False, True]),
            in_specs=[
                pl.BlockSpec((1, C, H, W), lambda i: (i, 0, 0, 0)),
                pl.BlockSpec((1, C, H, W), lambda i: (i, 0, 0, 0)),
                pl.BlockSpec((1, 1, H, W), lambda i: (i, 0, 0, 0)),
            ],
            out_specs=pl.BlockSpec(memory_space=pltpu.SMEM),
            out_shape=jax.ShapeDtypeStruct((2,), jnp.float32),
            scratch_shapes=[pltpu.SMEM((2,), jnp.float32)],
        )(pred, target, sky_mask.view(jnp.uint8))
    else:
        tc_parts = jnp.zeros((2,), jnp.float32)

    out = pl.pallas_call(
        _combine_body,
        in_specs=[
            pl.BlockSpec((_NW * 32,), lambda: (0,)),
            pl.BlockSpec(memory_space=pltpu.SMEM),
        ],
        out_specs=pl.BlockSpec(memory_space=pltpu.SMEM),
        out_shape=jax.ShapeDtypeStruct((1,), jnp.float32),
    )(sc_parts, tc_parts)
    return out[0]
